# baseline (speedup 1.0000x reference)
import functools

import jax
import jax.numpy as jnp
from jax import lax
from jax.experimental import pallas as pl
from jax.experimental.pallas import tpu as pltpu
from jax.experimental.pallas import tpu_sc as plsc

_DELTA_V = 0.5
_DELTA_D = 1.5
_ALPHA = 1.0
_BETA = 1.0
_GAMMA = 0.001
_K = 64
_L = 16
_NC = 2
_NS = 16
_B = 8
_N = 32768
_ROWS_PER_CORE = _B // _NC
_PTS = _N // _NS
_GRP = _PTS // _L
_CNT0 = _K * _L
_ACCW = 2 * _K * _L
_SLICE = _ACCW // _NS


def _iota16():
    return lax.iota(jnp.int32, _L)


def _rsqrt(x):
    i = plsc.bitcast(x, jnp.int32)
    i = 0x5F3759DF - (i >> 1)
    y = plsc.bitcast(i, jnp.float32)
    for _ in range(3):
        y = y * (1.5 - 0.5 * x * y * y)
    return y


def _sqrt(x):
    return x * _rsqrt(jnp.maximum(x, 1e-30))


def _sc_body(e_hbm, ids_hbm, out_hbm,
             ebuf, idbuf, accbuf, sumbuf, meansbuf, invbuf, redbuf, outbuf,
             shared_all, shared_red):
    c = lax.axis_index("c")
    s = lax.axis_index("s")
    wid = c * _NS + s
    lane = _iota16()
    zeros16 = jnp.zeros((_L,), jnp.float32)

    def row_body(r, carry):
        vacc, pairacc, regacc = carry
        row = c * _ROWS_PER_CORE + r

        def zero_acc(i, _):
            accbuf[pl.ds(i * _L, _L)] = zeros16
            return 0
        lax.fori_loop(0, _ACCW // _L, zero_acc, 0)

        pltpu.sync_copy(e_hbm.at[row, pl.ds(s * _PTS, _PTS), :], ebuf)
        pltpu.sync_copy(ids_hbm.at[row, pl.ds(s * _PTS, _PTS)], idbuf)

        def pa(g, _):
            idv = idbuf[pl.ds(g * _L, _L)]
            for j in range(_L):
                idj = idv[j]
                ev = ebuf[g * _L + j]
                off = idj * _L
                accbuf[pl.ds(off, _L)] = accbuf[pl.ds(off, _L)] + ev
                offc = _CNT0 + off
                accbuf[pl.ds(offc, _L)] = accbuf[pl.ds(offc, _L)] + 1.0
            return 0
        lax.fori_loop(0, _GRP, pa, 0)

        pltpu.sync_copy(accbuf, shared_all.at[s])
        plsc.subcore_barrier()
        pltpu.sync_copy(shared_all.at[:, pl.ds(s * _SLICE, _SLICE)], redbuf)
        for cc in range(_SLICE // _L):
            tot = redbuf[0, pl.ds(cc * _L, _L)]
            for t in range(1, _NS):
                tot = tot + redbuf[t, pl.ds(cc * _L, _L)]
            outbuf[...] = tot
            pltpu.sync_copy(
                outbuf, shared_red.at[pl.ds(s * _SLICE + cc * _L, _L)])
        plsc.subcore_barrier()
        pltpu.sync_copy(shared_red, sumbuf)
        plsc.subcore_barrier()

        def mk(k, _):
            cntv = sumbuf[pl.ds(_CNT0 + k * _L, _L)]
            ivv = 1.0 / jnp.maximum(cntv, 1.0)
            invbuf[pl.ds(k * _L, _L)] = ivv
            meansbuf[pl.ds(k * _L, _L)] = sumbuf[pl.ds(k * _L, _L)] * ivv
            return 0
        lax.fori_loop(0, _K, mk, 0)

        def pb(g, va):
            idv = idbuf[pl.ds(g * _L, _L)]
            ssqv = zeros16
            invgv = zeros16
            for j in range(_L):
                idj = idv[j]
                ev = ebuf[g * _L + j]
                mj = meansbuf[pl.ds(idj * _L, _L)]
                d = ev - mj
                ssqv = jnp.where(lane == j, jnp.sum(d * d), ssqv)
                invgv = jnp.where(
                    lane == j, invbuf[pl.ds(idj * _L, _L)], invgv)
            distv = _sqrt(ssqv + 1e-12)
            hv = jnp.maximum(distv - _DELTA_V, 0.0)
            return va + hv * hv * invgv
        vacc = lax.fori_loop(0, _GRP, pb, vacc)

        def pk(kk, pa_acc):
            k = s * 4 + kk
            mk_v = meansbuf[pl.ds(k * _L, _L)]

            def pjg(jg, acc):
                sqv = jnp.ones((_L,), jnp.float32)
                for j in range(_L):
                    mj = meansbuf[pl.ds(jg * _L * _L + j * _L, _L)]
                    d = mk_v - mj
                    sqv = jnp.where(lane == j, jnp.sum(d * d), sqv)
                pd = _sqrt(sqv)
                hp = jnp.maximum(2.0 * _DELTA_D - pd, 0.0)
                jidx = jg * _L + lane
                return acc + jnp.where(jidx == k, 0.0, hp * hp)
            return lax.fori_loop(0, _K // _L, pjg, pa_acc)
        pairacc = lax.fori_loop(0, 4, pk, pairacc)

        nsqv = jnp.ones((_L,), jnp.float32)
        for kk in range(4):
            mk_v = meansbuf[pl.ds((s * 4 + kk) * _L, _L)]
            nsqv = jnp.where(lane == kk, jnp.sum(mk_v * mk_v), nsqv)
        regacc = regacc + jnp.where(lane < 4, _sqrt(nsqv + 1e-12), 0.0)

        return (vacc, pairacc, regacc)

    init = (jnp.zeros((_L,), jnp.float32),) * 3
    vacc, pairacc, regacc = lax.fori_loop(0, _ROWS_PER_CORE, row_body, init)

    varp = jnp.sum(vacc)
    distp = jnp.sum(pairacc)
    regp = jnp.sum(regacc)
    outv = jnp.where(lane == 0, varp,
                     jnp.where(lane == 1, distp,
                               jnp.where(lane == 2, regp, 0.0)))
    outbuf[...] = outv
    pltpu.sync_copy(outbuf, out_hbm.at[wid])


@functools.partial(
    pl.kernel,
    mesh=plsc.VectorSubcoreMesh(core_axis_name="c", subcore_axis_name="s"),
    compiler_params=pltpu.CompilerParams(
        needs_layout_passes=False, use_tc_tiling_on_sc=False),
    out_type=jax.ShapeDtypeStruct((_NC * _NS, _L), jnp.float32),
    scratch_types=[
        pltpu.VMEM((_PTS, _L), jnp.float32),
        pltpu.VMEM((_PTS,), jnp.int32),
        pltpu.VMEM((_ACCW,), jnp.float32),
        pltpu.VMEM((_ACCW,), jnp.float32),
        pltpu.VMEM((_K * _L,), jnp.float32),
        pltpu.VMEM((_K * _L,), jnp.float32),
        pltpu.VMEM((_NS, _SLICE), jnp.float32),
        pltpu.VMEM((_L,), jnp.float32),
        pltpu.VMEM_SHARED((_NS, _ACCW), jnp.float32),
        pltpu.VMEM_SHARED((_ACCW,), jnp.float32),
    ],
)
def _sc_kernel(e_hbm, ids_hbm, out_hbm, *scratch):
    _sc_body(e_hbm, ids_hbm, out_hbm, *scratch)


@jax.jit
def kernel(embeddings, instance_ids):
    ids = instance_ids.astype(jnp.int32)
    p = _sc_kernel(embeddings, ids)
    num_pairs = _K * (_K - 1) / 2.0
    var_loss = jnp.sum(p[:, 0]) / (_K * _B)
    dist_loss = jnp.sum(p[:, 1]) / (2.0 * num_pairs * _B)
    reg_loss = jnp.sum(p[:, 2]) / (_K * _B)
    total = _ALPHA * var_loss + _BETA * dist_loss + _GAMMA * reg_loss
    return (total, var_loss, dist_loss, reg_loss)


# speedup vs baseline: 1.0680x; 1.0680x over previous
import functools

import jax
import jax.numpy as jnp
from jax import lax
from jax.experimental import pallas as pl
from jax.experimental.pallas import tpu as pltpu
from jax.experimental.pallas import tpu_sc as plsc

_DELTA_V = 0.5
_DELTA_D = 1.5
_ALPHA = 1.0
_BETA = 1.0
_GAMMA = 0.001
_K = 64
_L = 16
_NC = 2
_NS = 16
_B = 8
_N = 32768
_ROWS_PER_CORE = _B // _NC
_PTS = _N // _NS
_GRP = _PTS // _L
_CNT0 = _K * _L
_ACCW = 2 * _K * _L
_SLICE = _ACCW // _NS


def _iota16():
    return lax.iota(jnp.int32, _L)


def _rsqrt(x):
    i = plsc.bitcast(x, jnp.int32)
    i = 0x5F3759DF - (i >> 1)
    y = plsc.bitcast(i, jnp.float32)
    for _ in range(3):
        y = y * (1.5 - 0.5 * x * y * y)
    return y


def _sqrt(x):
    return x * _rsqrt(jnp.maximum(x, 1e-30))


def _sc_body(e_hbm, ids_hbm, out_hbm,
             ebuf, idbuf, accbuf, sumbuf, meansbuf, invbuf, redbuf, outbuf,
             shared_all, shared_red):
    c = lax.axis_index("c")
    s = lax.axis_index("s")
    wid = c * _NS + s
    lane = _iota16()
    zeros16 = jnp.zeros((_L,), jnp.float32)

    def row_body(r, carry):
        vacc, pairacc, regacc = carry
        row = c * _ROWS_PER_CORE + r

        def zero_acc(i, _):
            accbuf[pl.ds(i * _L, _L)] = zeros16
            return 0
        lax.fori_loop(0, _ACCW // _L, zero_acc, 0)

        pltpu.sync_copy(e_hbm.at[row, pl.ds(s * _PTS, _PTS), :], ebuf)
        pltpu.sync_copy(ids_hbm.at[row, pl.ds(s * _PTS, _PTS)], idbuf)

        rid0 = idbuf[pl.ds(0, _L)][0]

        def pa(g, st):
            racc, rcntv, rid = st
            base = g * _L
            idv = idbuf[pl.ds(base, _L)]
            uni = jnp.logical_and(idv[0] == rid, idv[_L - 1] == rid)

            def fast(racc, rcntv):
                t = [ebuf[base + j] for j in range(_L)]
                while len(t) > 1:
                    t = [t[i] + t[i + 1] for i in range(0, len(t), 2)]
                return racc + t[0], rcntv + 16.0, rid

            def slow(racc, rcntv):
                off = rid * _L
                accbuf[pl.ds(off, _L)] = accbuf[pl.ds(off, _L)] + racc
                offc = _CNT0 + off
                accbuf[pl.ds(offc, _L)] = accbuf[pl.ds(offc, _L)] + rcntv
                for j in range(_L):
                    o = idv[j] * _L
                    accbuf[pl.ds(o, _L)] = (
                        accbuf[pl.ds(o, _L)] + ebuf[base + j])
                    oc = _CNT0 + o
                    accbuf[pl.ds(oc, _L)] = accbuf[pl.ds(oc, _L)] + 1.0
                return zeros16, zeros16, idv[_L - 1]

            return lax.cond(uni, fast, slow, racc, rcntv)

        racc, rcntv, rid = lax.fori_loop(
            0, _GRP, pa, (zeros16, zeros16, rid0))
        off = rid * _L
        accbuf[pl.ds(off, _L)] = accbuf[pl.ds(off, _L)] + racc
        offc = _CNT0 + off
        accbuf[pl.ds(offc, _L)] = accbuf[pl.ds(offc, _L)] + rcntv

        pltpu.sync_copy(accbuf, shared_all.at[s])
        plsc.subcore_barrier()
        pltpu.sync_copy(shared_all.at[:, pl.ds(s * _SLICE, _SLICE)], redbuf)
        for cc in range(_SLICE // _L):
            tot = redbuf[0, pl.ds(cc * _L, _L)]
            for t in range(1, _NS):
                tot = tot + redbuf[t, pl.ds(cc * _L, _L)]
            outbuf[...] = tot
            pltpu.sync_copy(
                outbuf, shared_red.at[pl.ds(s * _SLICE + cc * _L, _L)])
        plsc.subcore_barrier()
        pltpu.sync_copy(shared_red, sumbuf)
        plsc.subcore_barrier()

        def mk(k, _):
            cntv = sumbuf[pl.ds(_CNT0 + k * _L, _L)]
            ivv = 1.0 / jnp.maximum(cntv, 1.0)
            invbuf[pl.ds(k * _L, _L)] = ivv
            meansbuf[pl.ds(k * _L, _L)] = sumbuf[pl.ds(k * _L, _L)] * ivv
            return 0
        lax.fori_loop(0, _K, mk, 0)

        def pb(g, st):
            va, mcur, icur, rid = st
            base = g * _L
            idv = idbuf[pl.ds(base, _L)]
            uni = jnp.logical_and(idv[0] == rid, idv[_L - 1] == rid)

            def fast(va, mcur, icur):
                ssqv = zeros16
                for j in range(_L):
                    d = ebuf[base + j] - mcur
                    ssqv = jnp.where(lane == j, jnp.sum(d * d), ssqv)
                distv = _sqrt(ssqv + 1e-12)
                hv = jnp.maximum(distv - _DELTA_V, 0.0)
                return va + hv * hv * icur, mcur, icur, rid

            def slow(va, mcur, icur):
                ssqv = zeros16
                invgv = zeros16
                for j in range(_L):
                    idj = idv[j]
                    d = ebuf[base + j] - meansbuf[pl.ds(idj * _L, _L)]
                    ssqv = jnp.where(lane == j, jnp.sum(d * d), ssqv)
                    invgv = jnp.where(
                        lane == j, invbuf[pl.ds(idj * _L, _L)], invgv)
                distv = _sqrt(ssqv + 1e-12)
                hv = jnp.maximum(distv - _DELTA_V, 0.0)
                nid = idv[_L - 1]
                return (va + hv * hv * invgv,
                        meansbuf[pl.ds(nid * _L, _L)],
                        invbuf[pl.ds(nid * _L, _L)], nid)

            return lax.cond(uni, fast, slow, va, mcur, icur)

        vacc, _, _, _ = lax.fori_loop(
            0, _GRP, pb,
            (vacc, meansbuf[pl.ds(rid0 * _L, _L)],
             invbuf[pl.ds(rid0 * _L, _L)], rid0))

        def pk(kk, pa_acc):
            k = s * 4 + kk
            mk_v = meansbuf[pl.ds(k * _L, _L)]

            def pjg(jg, acc):
                sqv = jnp.ones((_L,), jnp.float32)
                for j in range(_L):
                    mj = meansbuf[pl.ds(jg * _L * _L + j * _L, _L)]
                    d = mk_v - mj
                    sqv = jnp.where(lane == j, jnp.sum(d * d), sqv)
                pd = _sqrt(sqv)
                hp = jnp.maximum(2.0 * _DELTA_D - pd, 0.0)
                jidx = jg * _L + lane
                return acc + jnp.where(jidx == k, 0.0, hp * hp)
            return lax.fori_loop(0, _K // _L, pjg, pa_acc)
        pairacc = lax.fori_loop(0, 4, pk, pairacc)

        nsqv = jnp.ones((_L,), jnp.float32)
        for kk in range(4):
            mk_v = meansbuf[pl.ds((s * 4 + kk) * _L, _L)]
            nsqv = jnp.where(lane == kk, jnp.sum(mk_v * mk_v), nsqv)
        regacc = regacc + jnp.where(lane < 4, _sqrt(nsqv + 1e-12), 0.0)

        return (vacc, pairacc, regacc)

    init = (jnp.zeros((_L,), jnp.float32),) * 3
    vacc, pairacc, regacc = lax.fori_loop(0, _ROWS_PER_CORE, row_body, init)

    varp = jnp.sum(vacc)
    distp = jnp.sum(pairacc)
    regp = jnp.sum(regacc)
    outv = jnp.where(lane == 0, varp,
                     jnp.where(lane == 1, distp,
                               jnp.where(lane == 2, regp, 0.0)))
    outbuf[...] = outv
    pltpu.sync_copy(outbuf, out_hbm.at[wid])


@functools.partial(
    pl.kernel,
    mesh=plsc.VectorSubcoreMesh(core_axis_name="c", subcore_axis_name="s"),
    compiler_params=pltpu.CompilerParams(
        needs_layout_passes=False, use_tc_tiling_on_sc=False),
    out_type=jax.ShapeDtypeStruct((_NC * _NS, _L), jnp.float32),
    scratch_types=[
        pltpu.VMEM((_PTS, _L), jnp.float32),
        pltpu.VMEM((_PTS,), jnp.int32),
        pltpu.VMEM((_ACCW,), jnp.float32),
        pltpu.VMEM((_ACCW,), jnp.float32),
        pltpu.VMEM((_K * _L,), jnp.float32),
        pltpu.VMEM((_K * _L,), jnp.float32),
        pltpu.VMEM((_NS, _SLICE), jnp.float32),
        pltpu.VMEM((_L,), jnp.float32),
        pltpu.VMEM_SHARED((_NS, _ACCW), jnp.float32),
        pltpu.VMEM_SHARED((_ACCW,), jnp.float32),
    ],
)
def _sc_kernel(e_hbm, ids_hbm, out_hbm, *scratch):
    _sc_body(e_hbm, ids_hbm, out_hbm, *scratch)


@jax.jit
def kernel(embeddings, instance_ids):
    ids = instance_ids.astype(jnp.int32)
    p = _sc_kernel(embeddings, ids)
    num_pairs = _K * (_K - 1) / 2.0
    var_loss = jnp.sum(p[:, 0]) / (_K * _B)
    dist_loss = jnp.sum(p[:, 1]) / (2.0 * num_pairs * _B)
    reg_loss = jnp.sum(p[:, 2]) / (_K * _B)
    total = _ALPHA * var_loss + _BETA * dist_loss + _GAMMA * reg_loss
    return (total, var_loss, dist_loss, reg_loss)
